# baseline (speedup 1.0000x reference)
import functools

import jax
import jax.numpy as jnp
from jax import lax
from jax.experimental import pallas as pl
from jax.experimental.pallas import tpu as pltpu
from jax.experimental.pallas import tpu_sc as plsc

_B, _N, _D = 8, 8192, 64
_S, _K = 1024, 32


def _fps_body(x_ref, y_ref, z_ref, out_ref, dist_ref):
    b, n = x_ref.shape
    s = out_ref.shape[1]
    x = x_ref[...]
    y = y_ref[...]
    z = z_ref[...]
    dist_ref[...] = jnp.full((b, n), 1e10, jnp.float32)
    iota_n = jax.lax.broadcasted_iota(jnp.int32, (b, n), 1)
    iota_s = jax.lax.broadcasted_iota(jnp.int32, (b, s), 1)
    out_ref[...] = jnp.zeros((b, s), jnp.int32)

    def body(i, far):
        out_ref[...] = jnp.where(iota_s == i, far, out_ref[...])
        onehot = iota_n == far
        cx = jnp.sum(jnp.where(onehot, x, 0.0), axis=1, keepdims=True)
        cy = jnp.sum(jnp.where(onehot, y, 0.0), axis=1, keepdims=True)
        cz = jnp.sum(jnp.where(onehot, z, 0.0), axis=1, keepdims=True)
        dx = x - cx
        dy = y - cy
        dz = z - cz
        d = dx * dx + dy * dy + dz * dz
        dmin = jnp.minimum(dist_ref[...], d)
        dist_ref[...] = dmin
        m = jnp.max(dmin, axis=1, keepdims=True)
        far_new = jnp.min(
            jnp.where(dmin == m, iota_n, n), axis=1, keepdims=True
        ).astype(jnp.int32)
        return far_new

    jax.lax.fori_loop(0, s, body, jnp.zeros((b, 1), jnp.int32))


@functools.partial(jax.jit, static_argnames=("interpret",))
def _fps(xyz, interpret=False):
    b, n, _ = xyz.shape
    xt = jnp.transpose(xyz, (2, 0, 1))
    return pl.pallas_call(
        _fps_body,
        out_shape=jax.ShapeDtypeStruct((b, _S), jnp.int32),
        scratch_shapes=[pltpu.VMEM((b, n), jnp.float32)],
        interpret=interpret,
    )(xt[0], xt[1], xt[2])


_QT = 256


def _knn_body(qs_ref, k3_ref, out_ref, d_ref):
    qt, n = d_ref.shape
    k = out_ref.shape[2]
    qs = qs_ref[0]
    k3 = k3_ref[0]
    mm = jnp.dot(qs, k3, preferred_element_type=jnp.float32)
    qsq = jnp.sum(qs * qs, axis=1, keepdims=True)
    ksq = jnp.sum(k3 * k3, axis=0, keepdims=True)
    d_ref[...] = (-2.0 * mm + qsq) + ksq
    iota_n = jax.lax.broadcasted_iota(jnp.int32, (qt, n), 1)
    iota_k = jax.lax.broadcasted_iota(jnp.int32, (qt, k), 1)
    out_ref[...] = jnp.zeros((1, qt, k), jnp.int32)

    def body(j, if_prev):
        d = jnp.where(iota_n == if_prev, jnp.inf, d_ref[...])
        d_ref[...] = d
        m = jnp.min(d, axis=1, keepdims=True)
        ifound = jnp.min(
            jnp.where(d == m, iota_n, n), axis=1, keepdims=True
        ).astype(jnp.int32)
        out_ref[0] = jnp.where(iota_k == j, ifound, out_ref[0])
        return ifound

    jax.lax.fori_loop(0, k, body, jnp.full((qt, 1), -1, jnp.int32))


@functools.partial(jax.jit, static_argnames=("interpret",))
def _knn(xyz_sampled, xyz, interpret=False):
    b, n, _ = xyz.shape
    s = xyz_sampled.shape[1]
    ktr = jnp.transpose(xyz, (0, 2, 1))
    return pl.pallas_call(
        _knn_body,
        grid=(b, s // _QT),
        in_specs=[
            pl.BlockSpec((1, _QT, 3), lambda bb, ss: (bb, ss, 0)),
            pl.BlockSpec((1, 3, n), lambda bb, ss: (bb, 0, 0)),
        ],
        out_specs=pl.BlockSpec((1, _QT, _K), lambda bb, ss: (bb, ss, 0)),
        out_shape=jax.ShapeDtypeStruct((b, s, _K), jnp.int32),
        scratch_shapes=[pltpu.VMEM((_QT, n), jnp.float32)],
        interpret=interpret,
    )(xyz_sampled, ktr)


def _gather_rows(points, idx):
    b = points.shape[0]
    batch_idx = jnp.arange(b).reshape((b,) + (1,) * (idx.ndim - 1))
    return points[batch_idx, idx]


_NW = 32
_GW = 128
_CH = 128


def _sc_gather_fn(tab_hbm, gidx_hbm, sidx_hbm, rows_hbm, srows_hbm,
                  idx_v, buf_v, sidx_v, sbuf_v, sem):
    wid = lax.axis_index("s") * 2 + lax.axis_index("c")
    r_per_w = rows_hbm.shape[0] // _NW
    s_per_w = srows_hbm.shape[0] // _NW
    rbase = wid * r_per_w
    sbase = wid * s_per_w
    pltpu.sync_copy(gidx_hbm.at[pl.ds(rbase, r_per_w)], idx_v)
    pltpu.sync_copy(sidx_hbm.at[pl.ds(sbase, s_per_w)], sidx_v)

    def body(c, _):
        pltpu.async_copy(
            tab_hbm.at[idx_v.at[pl.ds(c * _CH, _CH)]], buf_v, sem).wait()
        pltpu.sync_copy(buf_v, rows_hbm.at[pl.ds(rbase + c * _CH, _CH)])
        return 0

    lax.fori_loop(0, r_per_w // _CH, body, 0)

    def sbody(c, _):
        pltpu.async_copy(
            tab_hbm.at[sidx_v.at[pl.ds(c * _CH, _CH)]], sbuf_v, sem).wait()
        pltpu.sync_copy(sbuf_v, srows_hbm.at[pl.ds(sbase + c * _CH, _CH)])
        return 0

    lax.fori_loop(0, s_per_w // _CH, sbody, 0)


@functools.partial(jax.jit, static_argnames=("interpret",))
def _sc_gather(tab, gidx, sidx, interpret=False):
    r = gidx.shape[0]
    ns = sidx.shape[0]
    mesh = plsc.VectorSubcoreMesh(core_axis_name="c", subcore_axis_name="s")
    return pl.kernel(
        _sc_gather_fn,
        out_type=[
            jax.ShapeDtypeStruct((r, _GW), jnp.float32),
            jax.ShapeDtypeStruct((ns, _GW), jnp.float32),
        ],
        mesh=mesh,
        scratch_types=[
            pltpu.VMEM((r // _NW,), jnp.int32),
            pltpu.VMEM((_CH, _GW), jnp.float32),
            pltpu.VMEM((ns // _NW,), jnp.int32),
            pltpu.VMEM((_CH, _GW), jnp.float32),
            pltpu.SemaphoreType.DMA,
        ],
        interpret=interpret,
    )(tab, gidx, sidx)


_ST = 32


def _stats_body(rows_ref, s1_ref, s2_ref):
    v = rows_ref[0]
    mean = jnp.mean(v, axis=1, keepdims=True)
    x = v - mean
    s1 = jnp.sum(x)
    s2 = jnp.sum(x * x)
    s1_ref[...] = jnp.full((1, 1, 1, 128), s1, jnp.float32)
    s2_ref[...] = jnp.full((1, 1, 1, 128), s2, jnp.float32)


def _finish_body(rows_ref, srows_ref, inv_ref, out_ref):
    v = rows_ref[0]
    fs = srows_ref[0]
    inv = inv_ref[pl.program_id(0)]
    mean = jnp.mean(v, axis=1, keepdims=True)
    xn = (v - mean)[:, :, : _D + 3] * inv
    fsb = jnp.broadcast_to(fs[:, None, :_D], (_ST, _K, _D))
    out_ref[0] = jnp.concatenate([xn, fsb], axis=-1)


@functools.partial(jax.jit, static_argnames=("interpret",))
def _normalize(rows4, srows3, interpret=False):
    b, s = rows4.shape[0], rows4.shape[1]
    nt = s // _ST
    s1, s2 = pl.pallas_call(
        _stats_body,
        grid=(b, nt),
        in_specs=[pl.BlockSpec((1, _ST, _K, _GW), lambda bb, tt: (bb, tt, 0, 0))],
        out_specs=[
            pl.BlockSpec((1, 1, 1, 128), lambda bb, tt: (bb, tt, 0, 0)),
            pl.BlockSpec((1, 1, 1, 128), lambda bb, tt: (bb, tt, 0, 0)),
        ],
        out_shape=[
            jax.ShapeDtypeStruct((b, nt, 1, 128), jnp.float32),
            jax.ShapeDtypeStruct((b, nt, 1, 128), jnp.float32),
        ],
        interpret=interpret,
    )(rows4)
    s1 = jnp.sum(s1[:, :, 0, 0], axis=1)
    s2 = jnp.sum(s2[:, :, 0, 0], axis=1)
    n = jnp.float32(s * _K * (_D + 3))
    mx = s1 / n
    var = (s2 - n * mx * mx) / (n - 1.0)
    inv = 1.0 / (jnp.sqrt(var) + 1e-05)
    f_out = pl.pallas_call(
        _finish_body,
        grid=(b, nt),
        in_specs=[
            pl.BlockSpec((1, _ST, _K, _GW), lambda bb, tt: (bb, tt, 0, 0)),
            pl.BlockSpec((1, _ST, _GW), lambda bb, tt: (bb, tt, 0)),
            pl.BlockSpec(memory_space=pltpu.SMEM),
        ],
        out_specs=pl.BlockSpec((1, _ST, _K, 2 * _D + 3),
                               lambda bb, tt: (bb, tt, 0, 0)),
        out_shape=jax.ShapeDtypeStruct((b, s, _K, 2 * _D + 3), jnp.float32),
        interpret=interpret,
    )(rows4, srows3, inv)
    return f_out


def kernel(xyz, f, affine_alpha, affine_beta):
    b, n, _ = xyz.shape
    idx = _fps(jax.lax.stop_gradient(xyz))
    xyz_sampled = _gather_rows(xyz, idx)
    knn_idx = _knn(jax.lax.stop_gradient(xyz_sampled),
                   jax.lax.stop_gradient(xyz))
    tab = jnp.concatenate(
        [f, xyz, jnp.zeros((b, n, _GW - _D - 3), jnp.float32)], axis=-1
    ).reshape(b * n, _GW)
    boff = (jnp.arange(b, dtype=jnp.int32) * n)
    gidx = (knn_idx + boff[:, None, None]).reshape(-1)
    sidx = (idx + boff[:, None]).reshape(-1)
    rows, srows = _sc_gather(tab, gidx, sidx)
    rows4 = rows.reshape(b, _S, _K, _GW)
    srows3 = srows.reshape(b, _S, _GW)
    f_out = _normalize(rows4, srows3)
    return (xyz_sampled, f_out)


# speedup vs baseline: 1.0662x; 1.0662x over previous
import functools

import jax
import jax.numpy as jnp
from jax import lax
from jax.experimental import pallas as pl
from jax.experimental.pallas import tpu as pltpu
from jax.experimental.pallas import tpu_sc as plsc

_B, _N, _D = 8, 8192, 64
_S, _K = 1024, 32


def _fps_body(x_ref, y_ref, z_ref, out_ref, dist_ref):
    b, n = x_ref.shape
    s = out_ref.shape[1]
    x = x_ref[...]
    y = y_ref[...]
    z = z_ref[...]
    dist_ref[...] = jnp.full((b, n), 1e10, jnp.float32)
    iota_n = jax.lax.broadcasted_iota(jnp.int32, (b, n), 1)
    iota_s = jax.lax.broadcasted_iota(jnp.int32, (b, s), 1)
    out_ref[...] = jnp.zeros((b, s), jnp.int32)

    def body(i, far):
        out_ref[...] = jnp.where(iota_s == i, far, out_ref[...])
        onehot = iota_n == far
        cx = jnp.sum(jnp.where(onehot, x, 0.0), axis=1, keepdims=True)
        cy = jnp.sum(jnp.where(onehot, y, 0.0), axis=1, keepdims=True)
        cz = jnp.sum(jnp.where(onehot, z, 0.0), axis=1, keepdims=True)
        dx = x - cx
        dy = y - cy
        dz = z - cz
        d = dx * dx + dy * dy + dz * dz
        dmin = jnp.minimum(dist_ref[...], d)
        dist_ref[...] = dmin
        m = jnp.max(dmin, axis=1, keepdims=True)
        far_new = jnp.min(
            jnp.where(dmin == m, iota_n, n), axis=1, keepdims=True
        ).astype(jnp.int32)
        return far_new

    jax.lax.fori_loop(0, s, body, jnp.zeros((b, 1), jnp.int32))


@functools.partial(jax.jit, static_argnames=("interpret",))
def _fps(xyz, interpret=False):
    b, n, _ = xyz.shape
    xt = jnp.transpose(xyz, (2, 0, 1))
    return pl.pallas_call(
        _fps_body,
        out_shape=jax.ShapeDtypeStruct((b, _S), jnp.int32),
        scratch_shapes=[pltpu.VMEM((b, n), jnp.float32)],
        interpret=interpret,
    )(xt[0], xt[1], xt[2])


_QT = 256


def _knn_body(qs_ref, k3_ref, out_ref, d_ref):
    qt, n = d_ref.shape
    k = out_ref.shape[2]
    qs = qs_ref[0]
    k3 = k3_ref[0]
    mm = jnp.dot(qs, k3, preferred_element_type=jnp.float32)
    qsq = jnp.sum(qs * qs, axis=1, keepdims=True)
    ksq = jnp.sum(k3 * k3, axis=0, keepdims=True)
    d_ref[...] = (-2.0 * mm + qsq) + ksq
    iota_n = jax.lax.broadcasted_iota(jnp.int32, (qt, n), 1)
    iota_k = jax.lax.broadcasted_iota(jnp.int32, (qt, k), 1)
    out_ref[...] = jnp.zeros((1, qt, k), jnp.int32)

    def body(j, _):
        d = d_ref[...]
        m = jnp.min(d, axis=1, keepdims=True)
        ifound = jnp.min(
            jnp.where(d == m, iota_n, n), axis=1, keepdims=True
        ).astype(jnp.int32)
        out_ref[0] = jnp.where(iota_k == j, ifound, out_ref[0])
        d_ref[...] = jnp.where(iota_n == ifound, jnp.inf, d)
        return 0

    jax.lax.fori_loop(0, k, body, 0)


@functools.partial(jax.jit, static_argnames=("interpret",))
def _knn(xyz_sampled, xyz, interpret=False):
    b, n, _ = xyz.shape
    s = xyz_sampled.shape[1]
    ktr = jnp.transpose(xyz, (0, 2, 1))
    return pl.pallas_call(
        _knn_body,
        grid=(b, s // _QT),
        in_specs=[
            pl.BlockSpec((1, _QT, 3), lambda bb, ss: (bb, ss, 0)),
            pl.BlockSpec((1, 3, n), lambda bb, ss: (bb, 0, 0)),
        ],
        out_specs=pl.BlockSpec((1, _QT, _K), lambda bb, ss: (bb, ss, 0)),
        out_shape=jax.ShapeDtypeStruct((b, s, _K), jnp.int32),
        scratch_shapes=[pltpu.VMEM((_QT, n), jnp.float32)],
        interpret=interpret,
    )(xyz_sampled, ktr)


def _gather_rows(points, idx):
    b = points.shape[0]
    batch_idx = jnp.arange(b).reshape((b,) + (1,) * (idx.ndim - 1))
    return points[batch_idx, idx]


_NW = 32
_GW = 128
_CH = 128


def _sc_gather_fn(tab_hbm, gidx_hbm, sidx_hbm, rows_hbm, srows_hbm,
                  idx_v, buf_v, sidx_v, sbuf_v, sem):
    wid = lax.axis_index("s") * 2 + lax.axis_index("c")
    r_per_w = rows_hbm.shape[0] // _NW
    s_per_w = srows_hbm.shape[0] // _NW
    rbase = wid * r_per_w
    sbase = wid * s_per_w
    pltpu.sync_copy(gidx_hbm.at[pl.ds(rbase, r_per_w)], idx_v)
    pltpu.sync_copy(sidx_hbm.at[pl.ds(sbase, s_per_w)], sidx_v)

    def body(c, _):
        pltpu.async_copy(
            tab_hbm.at[idx_v.at[pl.ds(c * _CH, _CH)]], buf_v, sem).wait()
        pltpu.sync_copy(buf_v, rows_hbm.at[pl.ds(rbase + c * _CH, _CH)])
        return 0

    lax.fori_loop(0, r_per_w // _CH, body, 0)

    def sbody(c, _):
        pltpu.async_copy(
            tab_hbm.at[sidx_v.at[pl.ds(c * _CH, _CH)]], sbuf_v, sem).wait()
        pltpu.sync_copy(sbuf_v, srows_hbm.at[pl.ds(sbase + c * _CH, _CH)])
        return 0

    lax.fori_loop(0, s_per_w // _CH, sbody, 0)


@functools.partial(jax.jit, static_argnames=("interpret",))
def _sc_gather(tab, gidx, sidx, interpret=False):
    r = gidx.shape[0]
    ns = sidx.shape[0]
    mesh = plsc.VectorSubcoreMesh(core_axis_name="c", subcore_axis_name="s")
    return pl.kernel(
        _sc_gather_fn,
        out_type=[
            jax.ShapeDtypeStruct((r, _GW), jnp.float32),
            jax.ShapeDtypeStruct((ns, _GW), jnp.float32),
        ],
        mesh=mesh,
        scratch_types=[
            pltpu.VMEM((r // _NW,), jnp.int32),
            pltpu.VMEM((_CH, _GW), jnp.float32),
            pltpu.VMEM((ns // _NW,), jnp.int32),
            pltpu.VMEM((_CH, _GW), jnp.float32),
            pltpu.SemaphoreType.DMA,
        ],
        interpret=interpret,
    )(tab, gidx, sidx)


_ST = 32


def _stats_body(rows_ref, s1_ref, s2_ref):
    v = rows_ref[0]
    mean = jnp.mean(v, axis=1, keepdims=True)
    x = v - mean
    s1 = jnp.sum(x)
    s2 = jnp.sum(x * x)
    s1_ref[...] = jnp.full((1, 1, 1, 128), s1, jnp.float32)
    s2_ref[...] = jnp.full((1, 1, 1, 128), s2, jnp.float32)


def _finish_body(rows_ref, srows_ref, inv_ref, out_ref):
    v = rows_ref[0]
    fs = srows_ref[0]
    inv = inv_ref[pl.program_id(0)]
    mean = jnp.mean(v, axis=1, keepdims=True)
    xn = (v - mean)[:, :, : _D + 3] * inv
    fsb = jnp.broadcast_to(fs[:, None, :_D], (_ST, _K, _D))
    out_ref[0] = jnp.concatenate([xn, fsb], axis=-1)


@functools.partial(jax.jit, static_argnames=("interpret",))
def _normalize(rows4, srows3, interpret=False):
    b, s = rows4.shape[0], rows4.shape[1]
    nt = s // _ST
    s1, s2 = pl.pallas_call(
        _stats_body,
        grid=(b, nt),
        in_specs=[pl.BlockSpec((1, _ST, _K, _GW), lambda bb, tt: (bb, tt, 0, 0))],
        out_specs=[
            pl.BlockSpec((1, 1, 1, 128), lambda bb, tt: (bb, tt, 0, 0)),
            pl.BlockSpec((1, 1, 1, 128), lambda bb, tt: (bb, tt, 0, 0)),
        ],
        out_shape=[
            jax.ShapeDtypeStruct((b, nt, 1, 128), jnp.float32),
            jax.ShapeDtypeStruct((b, nt, 1, 128), jnp.float32),
        ],
        interpret=interpret,
    )(rows4)
    s1 = jnp.sum(s1[:, :, 0, 0], axis=1)
    s2 = jnp.sum(s2[:, :, 0, 0], axis=1)
    n = jnp.float32(s * _K * (_D + 3))
    mx = s1 / n
    var = (s2 - n * mx * mx) / (n - 1.0)
    inv = 1.0 / (jnp.sqrt(var) + 1e-05)
    f_out = pl.pallas_call(
        _finish_body,
        grid=(b, nt),
        in_specs=[
            pl.BlockSpec((1, _ST, _K, _GW), lambda bb, tt: (bb, tt, 0, 0)),
            pl.BlockSpec((1, _ST, _GW), lambda bb, tt: (bb, tt, 0)),
            pl.BlockSpec(memory_space=pltpu.SMEM),
        ],
        out_specs=pl.BlockSpec((1, _ST, _K, 2 * _D + 3),
                               lambda bb, tt: (bb, tt, 0, 0)),
        out_shape=jax.ShapeDtypeStruct((b, s, _K, 2 * _D + 3), jnp.float32),
        interpret=interpret,
    )(rows4, srows3, inv)
    return f_out


def kernel(xyz, f, affine_alpha, affine_beta):
    b, n, _ = xyz.shape
    idx = _fps(jax.lax.stop_gradient(xyz))
    xyz_sampled = _gather_rows(xyz, idx)
    knn_idx = _knn(jax.lax.stop_gradient(xyz_sampled),
                   jax.lax.stop_gradient(xyz))
    tab = jnp.concatenate(
        [f, xyz, jnp.zeros((b, n, _GW - _D - 3), jnp.float32)], axis=-1
    ).reshape(b * n, _GW)
    boff = (jnp.arange(b, dtype=jnp.int32) * n)
    gidx = (knn_idx + boff[:, None, None]).reshape(-1)
    sidx = (idx + boff[:, None]).reshape(-1)
    rows, srows = _sc_gather(tab, gidx, sidx)
    rows4 = rows.reshape(b, _S, _K, _GW)
    srows3 = srows.reshape(b, _S, _GW)
    f_out = _normalize(rows4, srows3)
    return (xyz_sampled, f_out)


# speedup vs baseline: 1.1038x; 1.0352x over previous
import functools

import jax
import jax.numpy as jnp
from jax import lax
from jax.experimental import pallas as pl
from jax.experimental.pallas import tpu as pltpu
from jax.experimental.pallas import tpu_sc as plsc

_B, _N, _D = 8, 8192, 64
_S, _K = 1024, 32


def _fps_body(x_ref, y_ref, z_ref, out_ref, dist_ref):
    b, n = x_ref.shape
    s = out_ref.shape[1]
    x = x_ref[...]
    y = y_ref[...]
    z = z_ref[...]
    dist_ref[...] = jnp.full((b, n), 1e10, jnp.float32)
    iota_n = jax.lax.broadcasted_iota(jnp.int32, (b, n), 1)
    iota_s = jax.lax.broadcasted_iota(jnp.int32, (b, s), 1)
    out_ref[...] = jnp.zeros((b, s), jnp.int32)

    def body(i, far):
        out_ref[...] = jnp.where(iota_s == i, far, out_ref[...])
        onehot = iota_n == far
        cx = jnp.sum(jnp.where(onehot, x, 0.0), axis=1, keepdims=True)
        cy = jnp.sum(jnp.where(onehot, y, 0.0), axis=1, keepdims=True)
        cz = jnp.sum(jnp.where(onehot, z, 0.0), axis=1, keepdims=True)
        dx = x - cx
        dy = y - cy
        dz = z - cz
        d = dx * dx + dy * dy + dz * dz
        dmin = jnp.minimum(dist_ref[...], d)
        dist_ref[...] = dmin
        m = jnp.max(dmin, axis=1, keepdims=True)
        far_new = jnp.min(
            jnp.where(dmin == m, iota_n, n), axis=1, keepdims=True
        ).astype(jnp.int32)
        return far_new

    jax.lax.fori_loop(0, s, body, jnp.zeros((b, 1), jnp.int32))


@functools.partial(jax.jit, static_argnames=("interpret",))
def _fps(xyz, interpret=False):
    b, n, _ = xyz.shape
    xt = jnp.transpose(xyz, (2, 0, 1))
    return pl.pallas_call(
        _fps_body,
        out_shape=jax.ShapeDtypeStruct((b, _S), jnp.int32),
        scratch_shapes=[pltpu.VMEM((b, n), jnp.float32)],
        interpret=interpret,
    )(xt[0], xt[1], xt[2])


_QT = 512


def _knn_body(qs_ref, k3_ref, out_ref, d_ref):
    qt, n = d_ref.shape
    k = out_ref.shape[2]
    qs = qs_ref[0]
    k3 = k3_ref[0]
    mm = jnp.dot(qs, k3, preferred_element_type=jnp.float32)
    qsq = jnp.sum(qs * qs, axis=1, keepdims=True)
    ksq = jnp.sum(k3 * k3, axis=0, keepdims=True)
    d_ref[...] = (-2.0 * mm + qsq) + ksq
    iota_n = jax.lax.broadcasted_iota(jnp.int32, (qt, n), 1)
    iota_k = jax.lax.broadcasted_iota(jnp.int32, (qt, k), 1)
    out_ref[...] = jnp.zeros((1, qt, k), jnp.int32)

    def body(j, _):
        d = d_ref[...]
        m = jnp.min(d, axis=1, keepdims=True)
        ifound = jnp.min(
            jnp.where(d == m, iota_n, n), axis=1, keepdims=True
        ).astype(jnp.int32)
        out_ref[0] = jnp.where(iota_k == j, ifound, out_ref[0])
        d_ref[...] = jnp.where(iota_n == ifound, jnp.inf, d)
        return 0

    jax.lax.fori_loop(0, k, body, 0)


@functools.partial(jax.jit, static_argnames=("interpret",))
def _knn(xyz_sampled, xyz, interpret=False):
    b, n, _ = xyz.shape
    s = xyz_sampled.shape[1]
    ktr = jnp.transpose(xyz, (0, 2, 1))
    return pl.pallas_call(
        _knn_body,
        grid=(b, s // _QT),
        in_specs=[
            pl.BlockSpec((1, _QT, 3), lambda bb, ss: (bb, ss, 0)),
            pl.BlockSpec((1, 3, n), lambda bb, ss: (bb, 0, 0)),
        ],
        out_specs=pl.BlockSpec((1, _QT, _K), lambda bb, ss: (bb, ss, 0)),
        out_shape=jax.ShapeDtypeStruct((b, s, _K), jnp.int32),
        scratch_shapes=[pltpu.VMEM((_QT, n), jnp.float32)],
        interpret=interpret,
    )(xyz_sampled, ktr)


def _gather_rows(points, idx):
    b = points.shape[0]
    batch_idx = jnp.arange(b).reshape((b,) + (1,) * (idx.ndim - 1))
    return points[batch_idx, idx]


_NW = 32
_GW = 128
_CH = 128


def _sc_gather_fn(tab_hbm, gidx_hbm, sidx_hbm, rows_hbm, srows_hbm,
                  idx_v, buf_v, sidx_v, sbuf_v, sem):
    wid = lax.axis_index("s") * 2 + lax.axis_index("c")
    r_per_w = rows_hbm.shape[0] // _NW
    s_per_w = srows_hbm.shape[0] // _NW
    rbase = wid * r_per_w
    sbase = wid * s_per_w
    pltpu.sync_copy(gidx_hbm.at[pl.ds(rbase, r_per_w)], idx_v)
    pltpu.sync_copy(sidx_hbm.at[pl.ds(sbase, s_per_w)], sidx_v)

    def body(c, _):
        pltpu.async_copy(
            tab_hbm.at[idx_v.at[pl.ds(c * _CH, _CH)]], buf_v, sem).wait()
        pltpu.sync_copy(buf_v, rows_hbm.at[pl.ds(rbase + c * _CH, _CH)])
        return 0

    lax.fori_loop(0, r_per_w // _CH, body, 0)

    def sbody(c, _):
        pltpu.async_copy(
            tab_hbm.at[sidx_v.at[pl.ds(c * _CH, _CH)]], sbuf_v, sem).wait()
        pltpu.sync_copy(sbuf_v, srows_hbm.at[pl.ds(sbase + c * _CH, _CH)])
        return 0

    lax.fori_loop(0, s_per_w // _CH, sbody, 0)


@functools.partial(jax.jit, static_argnames=("interpret",))
def _sc_gather(tab, gidx, sidx, interpret=False):
    r = gidx.shape[0]
    ns = sidx.shape[0]
    mesh = plsc.VectorSubcoreMesh(core_axis_name="c", subcore_axis_name="s")
    return pl.kernel(
        _sc_gather_fn,
        out_type=[
            jax.ShapeDtypeStruct((r, _GW), jnp.float32),
            jax.ShapeDtypeStruct((ns, _GW), jnp.float32),
        ],
        mesh=mesh,
        scratch_types=[
            pltpu.VMEM((r // _NW,), jnp.int32),
            pltpu.VMEM((_CH, _GW), jnp.float32),
            pltpu.VMEM((ns // _NW,), jnp.int32),
            pltpu.VMEM((_CH, _GW), jnp.float32),
            pltpu.SemaphoreType.DMA,
        ],
        interpret=interpret,
    )(tab, gidx, sidx)


_ST = 32


def _stats_body(rows_ref, s1_ref, s2_ref):
    v = rows_ref[0]
    mean = jnp.mean(v, axis=1, keepdims=True)
    x = v - mean
    s1 = jnp.sum(x)
    s2 = jnp.sum(x * x)
    s1_ref[...] = jnp.full((1, 1, 1, 128), s1, jnp.float32)
    s2_ref[...] = jnp.full((1, 1, 1, 128), s2, jnp.float32)


def _finish_body(rows_ref, srows_ref, inv_ref, out_ref):
    v = rows_ref[0]
    fs = srows_ref[0]
    inv = inv_ref[pl.program_id(0)]
    mean = jnp.mean(v, axis=1, keepdims=True)
    xn = (v - mean)[:, :, : _D + 3] * inv
    fsb = jnp.broadcast_to(fs[:, None, :_D], (_ST, _K, _D))
    out_ref[0] = jnp.concatenate([xn, fsb], axis=-1)


@functools.partial(jax.jit, static_argnames=("interpret",))
def _normalize(rows4, srows3, interpret=False):
    b, s = rows4.shape[0], rows4.shape[1]
    nt = s // _ST
    s1, s2 = pl.pallas_call(
        _stats_body,
        grid=(b, nt),
        in_specs=[pl.BlockSpec((1, _ST, _K, _GW), lambda bb, tt: (bb, tt, 0, 0))],
        out_specs=[
            pl.BlockSpec((1, 1, 1, 128), lambda bb, tt: (bb, tt, 0, 0)),
            pl.BlockSpec((1, 1, 1, 128), lambda bb, tt: (bb, tt, 0, 0)),
        ],
        out_shape=[
            jax.ShapeDtypeStruct((b, nt, 1, 128), jnp.float32),
            jax.ShapeDtypeStruct((b, nt, 1, 128), jnp.float32),
        ],
        interpret=interpret,
    )(rows4)
    s1 = jnp.sum(s1[:, :, 0, 0], axis=1)
    s2 = jnp.sum(s2[:, :, 0, 0], axis=1)
    n = jnp.float32(s * _K * (_D + 3))
    mx = s1 / n
    var = (s2 - n * mx * mx) / (n - 1.0)
    inv = 1.0 / (jnp.sqrt(var) + 1e-05)
    f_out = pl.pallas_call(
        _finish_body,
        grid=(b, nt),
        in_specs=[
            pl.BlockSpec((1, _ST, _K, _GW), lambda bb, tt: (bb, tt, 0, 0)),
            pl.BlockSpec((1, _ST, _GW), lambda bb, tt: (bb, tt, 0)),
            pl.BlockSpec(memory_space=pltpu.SMEM),
        ],
        out_specs=pl.BlockSpec((1, _ST, _K, 2 * _D + 3),
                               lambda bb, tt: (bb, tt, 0, 0)),
        out_shape=jax.ShapeDtypeStruct((b, s, _K, 2 * _D + 3), jnp.float32),
        interpret=interpret,
    )(rows4, srows3, inv)
    return f_out


def kernel(xyz, f, affine_alpha, affine_beta):
    b, n, _ = xyz.shape
    idx = _fps(jax.lax.stop_gradient(xyz))
    xyz_sampled = _gather_rows(xyz, idx)
    knn_idx = _knn(jax.lax.stop_gradient(xyz_sampled),
                   jax.lax.stop_gradient(xyz))
    tab = jnp.concatenate(
        [f, xyz, jnp.zeros((b, n, _GW - _D - 3), jnp.float32)], axis=-1
    ).reshape(b * n, _GW)
    boff = (jnp.arange(b, dtype=jnp.int32) * n)
    gidx = (knn_idx + boff[:, None, None]).reshape(-1)
    sidx = (idx + boff[:, None]).reshape(-1)
    rows, srows = _sc_gather(tab, gidx, sidx)
    rows4 = rows.reshape(b, _S, _K, _GW)
    srows3 = srows.reshape(b, _S, _GW)
    f_out = _normalize(rows4, srows3)
    return (xyz_sampled, f_out)


# speedup vs baseline: 1.1202x; 1.0149x over previous
import functools

import jax
import jax.numpy as jnp
from jax import lax
from jax.experimental import pallas as pl
from jax.experimental.pallas import tpu as pltpu
from jax.experimental.pallas import tpu_sc as plsc

_B, _N, _D = 8, 8192, 64
_S, _K = 1024, 32


def _fps_body(x_ref, y_ref, z_ref, out_ref, dist_ref):
    b, n = x_ref.shape
    s = out_ref.shape[1]
    x = x_ref[...]
    y = y_ref[...]
    z = z_ref[...]
    dist_ref[...] = jnp.full((b, n), 1e10, jnp.float32)
    iota_n = jax.lax.broadcasted_iota(jnp.int32, (b, n), 1)
    iota_s = jax.lax.broadcasted_iota(jnp.int32, (b, s), 1)
    out_ref[...] = jnp.zeros((b, s), jnp.int32)

    def body(i, far):
        out_ref[...] = jnp.where(iota_s == i, far, out_ref[...])
        onehot = iota_n == far
        cx = jnp.sum(jnp.where(onehot, x, 0.0), axis=1, keepdims=True)
        cy = jnp.sum(jnp.where(onehot, y, 0.0), axis=1, keepdims=True)
        cz = jnp.sum(jnp.where(onehot, z, 0.0), axis=1, keepdims=True)
        dx = x - cx
        dy = y - cy
        dz = z - cz
        d = dx * dx + dy * dy + dz * dz
        dmin = jnp.minimum(dist_ref[...], d)
        dist_ref[...] = dmin
        m = jnp.max(dmin, axis=1, keepdims=True)
        far_new = jnp.min(
            jnp.where(dmin == m, iota_n, n), axis=1, keepdims=True
        ).astype(jnp.int32)
        return far_new

    jax.lax.fori_loop(0, s, body, jnp.zeros((b, 1), jnp.int32))


@functools.partial(jax.jit, static_argnames=("interpret",))
def _fps(xyz, interpret=False):
    b, n, _ = xyz.shape
    xt = jnp.transpose(xyz, (2, 0, 1))
    return pl.pallas_call(
        _fps_body,
        out_shape=jax.ShapeDtypeStruct((b, _S), jnp.int32),
        scratch_shapes=[pltpu.VMEM((b, n), jnp.float32)],
        interpret=interpret,
    )(xt[0], xt[1], xt[2])


_QT = 1024


def _knn_body(qs_ref, k3_ref, out_ref, d_ref):
    qt, n = d_ref.shape
    k = out_ref.shape[2]
    qs = qs_ref[0]
    k3 = k3_ref[0]
    mm = jnp.dot(qs, k3, preferred_element_type=jnp.float32)
    qsq = jnp.sum(qs * qs, axis=1, keepdims=True)
    ksq = jnp.sum(k3 * k3, axis=0, keepdims=True)
    d_ref[...] = (-2.0 * mm + qsq) + ksq
    iota_n = jax.lax.broadcasted_iota(jnp.int32, (qt, n), 1)
    iota_k = jax.lax.broadcasted_iota(jnp.int32, (qt, k), 1)
    out_ref[...] = jnp.zeros((1, qt, k), jnp.int32)

    def body(j, _):
        d = d_ref[...]
        m = jnp.min(d, axis=1, keepdims=True)
        ifound = jnp.min(
            jnp.where(d == m, iota_n, n), axis=1, keepdims=True
        ).astype(jnp.int32)
        out_ref[0] = jnp.where(iota_k == j, ifound, out_ref[0])
        d_ref[...] = jnp.where(iota_n == ifound, jnp.inf, d)
        return 0

    jax.lax.fori_loop(0, k, body, 0)


@functools.partial(jax.jit, static_argnames=("interpret",))
def _knn(xyz_sampled, xyz, interpret=False):
    b, n, _ = xyz.shape
    s = xyz_sampled.shape[1]
    ktr = jnp.transpose(xyz, (0, 2, 1))
    return pl.pallas_call(
        _knn_body,
        grid=(b, s // _QT),
        in_specs=[
            pl.BlockSpec((1, _QT, 3), lambda bb, ss: (bb, ss, 0)),
            pl.BlockSpec((1, 3, n), lambda bb, ss: (bb, 0, 0)),
        ],
        out_specs=pl.BlockSpec((1, _QT, _K), lambda bb, ss: (bb, ss, 0)),
        out_shape=jax.ShapeDtypeStruct((b, s, _K), jnp.int32),
        scratch_shapes=[pltpu.VMEM((_QT, n), jnp.float32)],
        interpret=interpret,
    )(xyz_sampled, ktr)


def _gather_rows(points, idx):
    b = points.shape[0]
    batch_idx = jnp.arange(b).reshape((b,) + (1,) * (idx.ndim - 1))
    return points[batch_idx, idx]


_NW = 32
_GW = 128
_CH = 128


def _sc_gather_fn(tab_hbm, gidx_hbm, sidx_hbm, rows_hbm, srows_hbm,
                  idx_v, buf_v, sidx_v, sbuf_v, sem):
    wid = lax.axis_index("s") * 2 + lax.axis_index("c")
    r_per_w = rows_hbm.shape[0] // _NW
    s_per_w = srows_hbm.shape[0] // _NW
    rbase = wid * r_per_w
    sbase = wid * s_per_w
    pltpu.sync_copy(gidx_hbm.at[pl.ds(rbase, r_per_w)], idx_v)
    pltpu.sync_copy(sidx_hbm.at[pl.ds(sbase, s_per_w)], sidx_v)

    def body(c, _):
        pltpu.async_copy(
            tab_hbm.at[idx_v.at[pl.ds(c * _CH, _CH)]], buf_v, sem).wait()
        pltpu.sync_copy(buf_v, rows_hbm.at[pl.ds(rbase + c * _CH, _CH)])
        return 0

    lax.fori_loop(0, r_per_w // _CH, body, 0)

    def sbody(c, _):
        pltpu.async_copy(
            tab_hbm.at[sidx_v.at[pl.ds(c * _CH, _CH)]], sbuf_v, sem).wait()
        pltpu.sync_copy(sbuf_v, srows_hbm.at[pl.ds(sbase + c * _CH, _CH)])
        return 0

    lax.fori_loop(0, s_per_w // _CH, sbody, 0)


@functools.partial(jax.jit, static_argnames=("interpret",))
def _sc_gather(tab, gidx, sidx, interpret=False):
    r = gidx.shape[0]
    ns = sidx.shape[0]
    mesh = plsc.VectorSubcoreMesh(core_axis_name="c", subcore_axis_name="s")
    return pl.kernel(
        _sc_gather_fn,
        out_type=[
            jax.ShapeDtypeStruct((r, _GW), jnp.float32),
            jax.ShapeDtypeStruct((ns, _GW), jnp.float32),
        ],
        mesh=mesh,
        scratch_types=[
            pltpu.VMEM((r // _NW,), jnp.int32),
            pltpu.VMEM((_CH, _GW), jnp.float32),
            pltpu.VMEM((ns // _NW,), jnp.int32),
            pltpu.VMEM((_CH, _GW), jnp.float32),
            pltpu.SemaphoreType.DMA,
        ],
        interpret=interpret,
    )(tab, gidx, sidx)


_ST = 32


def _stats_body(rows_ref, s1_ref, s2_ref):
    v = rows_ref[0]
    mean = jnp.mean(v, axis=1, keepdims=True)
    x = v - mean
    s1 = jnp.sum(x)
    s2 = jnp.sum(x * x)
    s1_ref[...] = jnp.full((1, 1, 1, 128), s1, jnp.float32)
    s2_ref[...] = jnp.full((1, 1, 1, 128), s2, jnp.float32)


def _finish_body(rows_ref, srows_ref, inv_ref, out_ref):
    v = rows_ref[0]
    fs = srows_ref[0]
    inv = inv_ref[pl.program_id(0)]
    mean = jnp.mean(v, axis=1, keepdims=True)
    xn = (v - mean)[:, :, : _D + 3] * inv
    fsb = jnp.broadcast_to(fs[:, None, :_D], (_ST, _K, _D))
    out_ref[0] = jnp.concatenate([xn, fsb], axis=-1)


@functools.partial(jax.jit, static_argnames=("interpret",))
def _normalize(rows4, srows3, interpret=False):
    b, s = rows4.shape[0], rows4.shape[1]
    nt = s // _ST
    s1, s2 = pl.pallas_call(
        _stats_body,
        grid=(b, nt),
        in_specs=[pl.BlockSpec((1, _ST, _K, _GW), lambda bb, tt: (bb, tt, 0, 0))],
        out_specs=[
            pl.BlockSpec((1, 1, 1, 128), lambda bb, tt: (bb, tt, 0, 0)),
            pl.BlockSpec((1, 1, 1, 128), lambda bb, tt: (bb, tt, 0, 0)),
        ],
        out_shape=[
            jax.ShapeDtypeStruct((b, nt, 1, 128), jnp.float32),
            jax.ShapeDtypeStruct((b, nt, 1, 128), jnp.float32),
        ],
        interpret=interpret,
    )(rows4)
    s1 = jnp.sum(s1[:, :, 0, 0], axis=1)
    s2 = jnp.sum(s2[:, :, 0, 0], axis=1)
    n = jnp.float32(s * _K * (_D + 3))
    mx = s1 / n
    var = (s2 - n * mx * mx) / (n - 1.0)
    inv = 1.0 / (jnp.sqrt(var) + 1e-05)
    f_out = pl.pallas_call(
        _finish_body,
        grid=(b, nt),
        in_specs=[
            pl.BlockSpec((1, _ST, _K, _GW), lambda bb, tt: (bb, tt, 0, 0)),
            pl.BlockSpec((1, _ST, _GW), lambda bb, tt: (bb, tt, 0)),
            pl.BlockSpec(memory_space=pltpu.SMEM),
        ],
        out_specs=pl.BlockSpec((1, _ST, _K, 2 * _D + 3),
                               lambda bb, tt: (bb, tt, 0, 0)),
        out_shape=jax.ShapeDtypeStruct((b, s, _K, 2 * _D + 3), jnp.float32),
        interpret=interpret,
    )(rows4, srows3, inv)
    return f_out


def kernel(xyz, f, affine_alpha, affine_beta):
    b, n, _ = xyz.shape
    idx = _fps(jax.lax.stop_gradient(xyz))
    xyz_sampled = _gather_rows(xyz, idx)
    knn_idx = _knn(jax.lax.stop_gradient(xyz_sampled),
                   jax.lax.stop_gradient(xyz))
    tab = jnp.concatenate(
        [f, xyz, jnp.zeros((b, n, _GW - _D - 3), jnp.float32)], axis=-1
    ).reshape(b * n, _GW)
    boff = (jnp.arange(b, dtype=jnp.int32) * n)
    gidx = (knn_idx + boff[:, None, None]).reshape(-1)
    sidx = (idx + boff[:, None]).reshape(-1)
    rows, srows = _sc_gather(tab, gidx, sidx)
    rows4 = rows.reshape(b, _S, _K, _GW)
    srows3 = srows.reshape(b, _S, _GW)
    f_out = _normalize(rows4, srows3)
    return (xyz_sampled, f_out)


# speedup vs baseline: 1.1634x; 1.0386x over previous
import functools

import jax
import jax.numpy as jnp
from jax import lax
from jax.experimental import pallas as pl
from jax.experimental.pallas import tpu as pltpu
from jax.experimental.pallas import tpu_sc as plsc

_B, _N, _D = 8, 8192, 64
_S, _K = 1024, 32


def _fps_body(x_ref, y_ref, z_ref, out_ref, dist_ref):
    b, n = x_ref.shape
    s = out_ref.shape[1]
    x = x_ref[...]
    y = y_ref[...]
    z = z_ref[...]
    dist_ref[...] = jnp.full((b, n), 1e10, jnp.float32)
    iota_n = jax.lax.broadcasted_iota(jnp.int32, (b, n), 1)
    iota_s = jax.lax.broadcasted_iota(jnp.int32, (b, s), 1)
    out_ref[...] = jnp.zeros((b, s), jnp.int32)

    def body(i, far):
        out_ref[...] = jnp.where(iota_s == i, far, out_ref[...])
        onehot = iota_n == far
        cx = jnp.sum(jnp.where(onehot, x, 0.0), axis=1, keepdims=True)
        cy = jnp.sum(jnp.where(onehot, y, 0.0), axis=1, keepdims=True)
        cz = jnp.sum(jnp.where(onehot, z, 0.0), axis=1, keepdims=True)
        dx = x - cx
        dy = y - cy
        dz = z - cz
        d = dx * dx + dy * dy + dz * dz
        dmin = jnp.minimum(dist_ref[...], d)
        dist_ref[...] = dmin
        m = jnp.max(dmin, axis=1, keepdims=True)
        far_new = jnp.min(
            jnp.where(dmin == m, iota_n, n), axis=1, keepdims=True
        ).astype(jnp.int32)
        return far_new

    jax.lax.fori_loop(0, s, body, jnp.zeros((b, 1), jnp.int32))


@functools.partial(jax.jit, static_argnames=("interpret",))
def _fps(xyz, interpret=False):
    b, n, _ = xyz.shape
    xt = jnp.transpose(xyz, (2, 0, 1))
    return pl.pallas_call(
        _fps_body,
        out_shape=jax.ShapeDtypeStruct((b, _S), jnp.int32),
        scratch_shapes=[pltpu.VMEM((b, n), jnp.float32)],
        interpret=interpret,
    )(xt[0], xt[1], xt[2])


_QT = 1024


def _knn_body(qs_ref, k3_ref, out_ref, d_ref):
    qt, n = d_ref.shape
    k = out_ref.shape[2]
    qs = qs_ref[0]
    k3 = k3_ref[0]
    mm = jnp.dot(qs, k3, preferred_element_type=jnp.float32)
    qsq = jnp.sum(qs * qs, axis=1, keepdims=True)
    ksq = jnp.sum(k3 * k3, axis=0, keepdims=True)
    d_ref[...] = (-2.0 * mm + qsq) + ksq
    iota_n = jax.lax.broadcasted_iota(jnp.int32, (qt, n), 1)
    iota_k = jax.lax.broadcasted_iota(jnp.int32, (qt, k), 1)
    out_ref[...] = jnp.zeros((1, qt, k), jnp.int32)

    def body(j, _):
        d = d_ref[...]
        m = jnp.min(d, axis=1, keepdims=True)
        ifound = jnp.min(
            jnp.where(d == m, iota_n, n), axis=1, keepdims=True
        ).astype(jnp.int32)
        out_ref[0] = jnp.where(iota_k == j, ifound, out_ref[0])
        d_ref[...] = jnp.where(iota_n == ifound, jnp.inf, d)
        return 0

    jax.lax.fori_loop(0, k, body, 0)


@functools.partial(jax.jit, static_argnames=("interpret",))
def _knn(xyz_sampled, xyz, interpret=False):
    b, n, _ = xyz.shape
    s = xyz_sampled.shape[1]
    ktr = jnp.transpose(xyz, (0, 2, 1))
    return pl.pallas_call(
        _knn_body,
        grid=(b, s // _QT),
        in_specs=[
            pl.BlockSpec((1, _QT, 3), lambda bb, ss: (bb, ss, 0)),
            pl.BlockSpec((1, 3, n), lambda bb, ss: (bb, 0, 0)),
        ],
        out_specs=pl.BlockSpec((1, _QT, _K), lambda bb, ss: (bb, ss, 0)),
        out_shape=jax.ShapeDtypeStruct((b, s, _K), jnp.int32),
        scratch_shapes=[pltpu.VMEM((_QT, n), jnp.float32)],
        interpret=interpret,
    )(xyz_sampled, ktr)


def _gather_rows(points, idx):
    b = points.shape[0]
    batch_idx = jnp.arange(b).reshape((b,) + (1,) * (idx.ndim - 1))
    return points[batch_idx, idx]


_NW = 32
_GW = 128
_CH = 128


def _sc_gather_fn(tab_hbm, gidx_hbm, sidx_hbm, rows_hbm, srows_hbm,
                  idx_v, buf_v, sidx_v, sbuf_v, sem):
    wid = lax.axis_index("s") * 2 + lax.axis_index("c")
    r_per_w = rows_hbm.shape[0] // _NW
    s_per_w = srows_hbm.shape[0] // _NW
    rbase = wid * r_per_w
    sbase = wid * s_per_w
    pltpu.sync_copy(gidx_hbm.at[pl.ds(rbase, r_per_w)], idx_v)
    pltpu.sync_copy(sidx_hbm.at[pl.ds(sbase, s_per_w)], sidx_v)

    def body(c, _):
        pltpu.async_copy(
            tab_hbm.at[idx_v.at[pl.ds(c * _CH, _CH)]], buf_v, sem).wait()
        pltpu.sync_copy(buf_v, rows_hbm.at[pl.ds(rbase + c * _CH, _CH)])
        return 0

    lax.fori_loop(0, r_per_w // _CH, body, 0)

    def sbody(c, _):
        pltpu.async_copy(
            tab_hbm.at[sidx_v.at[pl.ds(c * _CH, _CH)]], sbuf_v, sem).wait()
        pltpu.sync_copy(sbuf_v, srows_hbm.at[pl.ds(sbase + c * _CH, _CH)])
        return 0

    lax.fori_loop(0, s_per_w // _CH, sbody, 0)


@functools.partial(jax.jit, static_argnames=("interpret",))
def _sc_gather(tab, gidx, sidx, interpret=False):
    r = gidx.shape[0]
    ns = sidx.shape[0]
    mesh = plsc.VectorSubcoreMesh(core_axis_name="c", subcore_axis_name="s")
    return pl.kernel(
        _sc_gather_fn,
        out_type=[
            jax.ShapeDtypeStruct((r, _GW), jnp.float32),
            jax.ShapeDtypeStruct((ns, _GW), jnp.float32),
        ],
        mesh=mesh,
        scratch_types=[
            pltpu.VMEM((r // _NW,), jnp.int32),
            pltpu.VMEM((_CH, _GW), jnp.float32),
            pltpu.VMEM((ns // _NW,), jnp.int32),
            pltpu.VMEM((_CH, _GW), jnp.float32),
            pltpu.SemaphoreType.DMA,
        ],
        interpret=interpret,
    )(tab, gidx, sidx)


_ST = 64


def _stats_body(rows_ref, s1_ref, s2_ref):
    v = rows_ref[0]
    mean = jnp.mean(v, axis=1, keepdims=True)
    x = v - mean
    s1 = jnp.sum(x)
    s2 = jnp.sum(x * x)
    s1_ref[...] = jnp.full((1, 1, 1, 128), s1, jnp.float32)
    s2_ref[...] = jnp.full((1, 1, 1, 128), s2, jnp.float32)


def _finish_body(rows_ref, srows_ref, inv_ref, out_ref):
    v = rows_ref[0]
    fs = srows_ref[0]
    inv = inv_ref[pl.program_id(0)]
    mean = jnp.mean(v, axis=1, keepdims=True)
    xn = (v - mean)[:, :, : _D + 3] * inv
    fsb = jnp.broadcast_to(fs[:, None, :_D], (_ST, _K, _D))
    out_ref[0] = jnp.concatenate([xn, fsb], axis=-1)


@functools.partial(jax.jit, static_argnames=("interpret",))
def _normalize(rows4, srows3, interpret=False):
    b, s = rows4.shape[0], rows4.shape[1]
    nt = s // _ST
    s1, s2 = pl.pallas_call(
        _stats_body,
        grid=(b, nt),
        in_specs=[pl.BlockSpec((1, _ST, _K, _GW), lambda bb, tt: (bb, tt, 0, 0))],
        out_specs=[
            pl.BlockSpec((1, 1, 1, 128), lambda bb, tt: (bb, tt, 0, 0)),
            pl.BlockSpec((1, 1, 1, 128), lambda bb, tt: (bb, tt, 0, 0)),
        ],
        out_shape=[
            jax.ShapeDtypeStruct((b, nt, 1, 128), jnp.float32),
            jax.ShapeDtypeStruct((b, nt, 1, 128), jnp.float32),
        ],
        interpret=interpret,
    )(rows4)
    s1 = jnp.sum(s1[:, :, 0, 0], axis=1)
    s2 = jnp.sum(s2[:, :, 0, 0], axis=1)
    n = jnp.float32(s * _K * (_D + 3))
    mx = s1 / n
    var = (s2 - n * mx * mx) / (n - 1.0)
    inv = 1.0 / (jnp.sqrt(var) + 1e-05)
    f_out = pl.pallas_call(
        _finish_body,
        grid=(b, nt),
        in_specs=[
            pl.BlockSpec((1, _ST, _K, _GW), lambda bb, tt: (bb, tt, 0, 0)),
            pl.BlockSpec((1, _ST, _GW), lambda bb, tt: (bb, tt, 0)),
            pl.BlockSpec(memory_space=pltpu.SMEM),
        ],
        out_specs=pl.BlockSpec((1, _ST, _K, 2 * _D + 3),
                               lambda bb, tt: (bb, tt, 0, 0)),
        out_shape=jax.ShapeDtypeStruct((b, s, _K, 2 * _D + 3), jnp.float32),
        interpret=interpret,
    )(rows4, srows3, inv)
    return f_out


def kernel(xyz, f, affine_alpha, affine_beta):
    b, n, _ = xyz.shape
    idx = _fps(jax.lax.stop_gradient(xyz))
    xyz_sampled = _gather_rows(xyz, idx)
    knn_idx = _knn(jax.lax.stop_gradient(xyz_sampled),
                   jax.lax.stop_gradient(xyz))
    tab = jnp.concatenate(
        [f, xyz, jnp.zeros((b, n, _GW - _D - 3), jnp.float32)], axis=-1
    ).reshape(b * n, _GW)
    boff = (jnp.arange(b, dtype=jnp.int32) * n)
    gidx = (knn_idx + boff[:, None, None]).reshape(-1)
    sidx = (idx + boff[:, None]).reshape(-1)
    rows, srows = _sc_gather(tab, gidx, sidx)
    rows4 = rows.reshape(b, _S, _K, _GW)
    srows3 = srows.reshape(b, _S, _GW)
    f_out = _normalize(rows4, srows3)
    return (xyz_sampled, f_out)
